# Initial kernel scaffold; baseline (speedup 1.0000x reference)
import jax
import jax.numpy as jnp
from jax.experimental import pallas as pl


def kernel(indices, table):
    raise NotImplementedError("write your pallas kernel here")


# speedup vs baseline: 3.5706x; 3.5706x over previous
import functools

import jax
import jax.numpy as jnp
from jax import lax
from jax.experimental import pallas as pl
from jax.experimental.pallas import tpu as pltpu
from jax.experimental.pallas import tpu_sc as plsc

_INFO = plsc.get_sparse_core_info()
_NC = _INFO.num_cores
_NS = _INFO.num_subcores
_NW = _NC * _NS

_CHUNK = 128


def _make_gather(n_rows, vocab, dim):
    assert n_rows % _NW == 0
    b_per_w = n_rows // _NW
    assert b_per_w % _CHUNK == 0
    n_chunks = b_per_w // _CHUNK

    @functools.partial(
        pl.kernel,
        mesh=plsc.VectorSubcoreMesh(core_axis_name="c", subcore_axis_name="s"),
        out_type=jax.ShapeDtypeStruct((n_rows, dim), jnp.float32),
        scratch_types=[
            pltpu.VMEM((b_per_w,), jnp.int32),
            pltpu.VMEM((_CHUNK, dim), jnp.float32),
            pltpu.SemaphoreType.DMA,
        ],
        compiler_params=pltpu.CompilerParams(use_tc_tiling_on_sc=False),
    )
    def gather_kernel(table_hbm, idx_hbm, out_hbm, idx_v, rows_v, sem):
        wid = lax.axis_index("s") * _NC + lax.axis_index("c")
        base = pl.multiple_of(wid * b_per_w, _CHUNK)
        pltpu.sync_copy(idx_hbm.at[pl.ds(base, b_per_w)], idx_v)

        def body(i, carry):
            off = pl.multiple_of(i * _CHUNK, _CHUNK)
            pltpu.async_copy(
                table_hbm.at[idx_v.at[pl.ds(off, _CHUNK)]], rows_v, sem
            ).wait()
            pltpu.sync_copy(rows_v, out_hbm.at[pl.ds(base + off, _CHUNK)])
            return carry

        lax.fori_loop(0, n_chunks, body, 0)

    return gather_kernel


def kernel(indices, table):
    batch, hist = indices.shape
    vocab, dim = table.shape
    n_rows = batch * hist
    idx_flat = indices.reshape(n_rows)
    out = _make_gather(n_rows, vocab, dim)(table, idx_flat)
    return out.reshape(batch, hist, dim)


# speedup vs baseline: 3.5888x; 1.0051x over previous
import functools

import jax
import jax.numpy as jnp
from jax import lax
from jax.experimental import pallas as pl
from jax.experimental.pallas import tpu as pltpu
from jax.experimental.pallas import tpu_sc as plsc

_INFO = plsc.get_sparse_core_info()
_NC = _INFO.num_cores
_NS = _INFO.num_subcores
_NW = _NC * _NS

_CHUNK = 128
_K = 2
_ROWS = _K * _CHUNK
_NBUF = 4


def _make_gather(n_rows, vocab, dim):
    assert n_rows % _NW == 0
    b_per_w = n_rows // _NW
    assert b_per_w % (_ROWS * _NBUF) == 0
    n_it = b_per_w // _ROWS
    n_groups = n_it // _NBUF

    @functools.partial(
        pl.kernel,
        mesh=plsc.VectorSubcoreMesh(core_axis_name="c", subcore_axis_name="s"),
        out_type=jax.ShapeDtypeStruct((n_rows, dim), jnp.float32),
        scratch_types=[
            pltpu.VMEM((b_per_w,), jnp.int32),
            pltpu.VMEM((_NBUF, _ROWS, dim), jnp.float32),
        ]
        + [pltpu.SemaphoreType.DMA] * (2 * _NBUF),
        compiler_params=pltpu.CompilerParams(use_tc_tiling_on_sc=False),
    )
    def gather_kernel(table_hbm, idx_hbm, out_hbm, idx_v, rows, *sems):
        gsem = sems[:_NBUF]
        ssem = sems[_NBUF:]
        wid = lax.axis_index("s") * _NC + lax.axis_index("c")
        base = pl.multiple_of(wid * b_per_w, _ROWS)
        pltpu.sync_copy(idx_hbm.at[pl.ds(base, b_per_w)], idx_v)

        def fire_gather(i, b):
            off = pl.multiple_of(i * _ROWS, _ROWS)
            for k in range(_K):
                pltpu.async_copy(
                    table_hbm.at[idx_v.at[pl.ds(off + k * _CHUNK, _CHUNK)]],
                    rows.at[b].at[pl.ds(k * _CHUNK, _CHUNK)],
                    gsem[b],
                )

        def drain_gather(b):
            pltpu.make_async_copy(
                out_hbm.at[pl.ds(0, _ROWS)], rows.at[b], gsem[b]
            ).wait()

        def fire_store(i, b):
            off = pl.multiple_of(i * _ROWS, _ROWS)
            pltpu.async_copy(rows.at[b], out_hbm.at[pl.ds(base + off, _ROWS)], ssem[b])

        def wait_store(b):
            pltpu.make_async_copy(
                out_hbm.at[pl.ds(0, _ROWS)], rows.at[b], ssem[b]
            ).wait()

        for b in range(_NBUF):
            fire_gather(b, b)

        def body(g, carry):
            for b in range(_NBUF):
                i = g * _NBUF + b
                drain_gather(b)
                fire_store(i, b)

                @pl.when(g < n_groups - 1)
                def _():
                    wait_store(b)
                    fire_gather(i + _NBUF, b)

            return carry

        lax.fori_loop(0, n_groups, body, 0)
        for b in range(_NBUF):
            wait_store(b)

    return gather_kernel


def kernel(indices, table):
    batch, hist = indices.shape
    vocab, dim = table.shape
    n_rows = batch * hist
    idx_flat = indices.reshape(n_rows)
    out = _make_gather(n_rows, vocab, dim)(table, idx_flat)
    return out.reshape(batch, hist, dim)
